# Initial kernel scaffold; baseline (speedup 1.0000x reference)
import jax
import jax.numpy as jnp
from jax.experimental import pallas as pl


def kernel(input_ids, complement_map, weight):
    raise NotImplementedError("write your pallas kernel here")


# speedup vs baseline: 4.3567x; 4.3567x over previous
import functools

import jax
import jax.numpy as jnp
from jax import lax
from jax.experimental import pallas as pl
from jax.experimental.pallas import tpu as pltpu
from jax.experimental.pallas import tpu_sc as plsc

NC = 2
NS = 16
LANES = 16
NW = NC * NS

VOCAB = 16
D_MODEL = 256
D_OUT = 2 * D_MODEL


def _build_table(weight, cmap):
    mesh = plsc.VectorSubcoreMesh(core_axis_name="c", subcore_axis_name="s")

    @functools.partial(
        pl.kernel,
        mesh=mesh,
        out_type=jax.ShapeDtypeStruct((VOCAB, D_OUT), jnp.float32),
        scratch_types=[
            pltpu.VMEM((VOCAB,), jnp.int32),
            pltpu.VMEM((VOCAB, D_MODEL), jnp.float32),
            pltpu.VMEM((VOCAB, D_MODEL), jnp.float32),
            pltpu.VMEM((VOCAB, D_OUT), jnp.float32),
            pltpu.SemaphoreType.DMA,
        ],
    )
    def build(weight_hbm, cmap_hbm, tbl_hbm, cmap_v, wv, crows, tbl, sem):
        wid = lax.axis_index("s") * NC + lax.axis_index("c")

        @pl.when(wid == 0)
        def _():
            pltpu.sync_copy(weight_hbm, wv)
            pltpu.sync_copy(cmap_hbm, cmap_v)
            pltpu.async_copy(weight_hbm.at[cmap_v], crows, sem).wait()
            for v in range(VOCAB):
                for j in range(D_MODEL // LANES):
                    tbl[v, pl.ds(j * LANES, LANES)] = wv[v, pl.ds(j * LANES, LANES)]
                    rev = lax.rev(
                        crows[v, pl.ds(D_MODEL - (j + 1) * LANES, LANES)],
                        dimensions=(0,),
                    )
                    tbl[v, pl.ds(D_MODEL + j * LANES, LANES)] = rev
            pltpu.sync_copy(tbl, tbl_hbm)

    return build(weight, cmap)


def _lookup(table, ids_flat, n_tokens):
    t_per_w = n_tokens // NW
    chunk = 128
    n_chunks = t_per_w // chunk
    mesh = plsc.VectorSubcoreMesh(core_axis_name="c", subcore_axis_name="s")

    @functools.partial(
        pl.kernel,
        mesh=mesh,
        out_type=jax.ShapeDtypeStruct((n_tokens, D_OUT), jnp.float32),
        scratch_types=[
            pltpu.VMEM((t_per_w,), jnp.int32),
            pltpu.VMEM((chunk, D_OUT), jnp.float32),
            pltpu.SemaphoreType.DMA,
        ],
    )
    def look(tbl_hbm, ids_hbm, out_hbm, idx_v, rows_v, sem):
        wid = lax.axis_index("s") * NC + lax.axis_index("c")
        base = wid * t_per_w
        pltpu.sync_copy(ids_hbm.at[pl.ds(base, t_per_w)], idx_v)

        @pl.loop(0, n_chunks)
        def _(c):
            pltpu.async_copy(
                tbl_hbm.at[idx_v.at[pl.ds(c * chunk, chunk)]], rows_v, sem
            ).wait()
            pltpu.sync_copy(rows_v, out_hbm.at[pl.ds(base + c * chunk, chunk)])

    return look(table, ids_flat)


def kernel(input_ids, complement_map, weight):
    b, l = input_ids.shape
    n_tokens = b * l
    ids_flat = input_ids.reshape(n_tokens)
    table = _build_table(weight, complement_map)
    out = _lookup(table, ids_flat, n_tokens)
    return out.reshape(b, l, D_OUT)


# speedup vs baseline: 4.4058x; 1.0113x over previous
import functools

import jax
import jax.numpy as jnp
from jax import lax
from jax.experimental import pallas as pl
from jax.experimental.pallas import tpu as pltpu
from jax.experimental.pallas import tpu_sc as plsc

NC = 2
NS = 16
LANES = 16
NW = NC * NS

VOCAB = 16
D_MODEL = 256
D_OUT = 2 * D_MODEL


def _build_table(weight, cmap):
    mesh = plsc.VectorSubcoreMesh(core_axis_name="c", subcore_axis_name="s")

    @functools.partial(
        pl.kernel,
        mesh=mesh,
        out_type=jax.ShapeDtypeStruct((VOCAB, D_OUT), jnp.float32),
        scratch_types=[
            pltpu.VMEM((VOCAB,), jnp.int32),
            pltpu.VMEM((VOCAB, D_MODEL), jnp.float32),
            pltpu.VMEM((VOCAB, D_MODEL), jnp.float32),
            pltpu.VMEM((VOCAB, D_OUT), jnp.float32),
            pltpu.SemaphoreType.DMA,
        ],
    )
    def build(weight_hbm, cmap_hbm, tbl_hbm, cmap_v, wv, crows, tbl, sem):
        wid = lax.axis_index("s") * NC + lax.axis_index("c")

        @pl.when(wid == 0)
        def _():
            pltpu.sync_copy(weight_hbm, wv)
            pltpu.sync_copy(cmap_hbm, cmap_v)
            pltpu.async_copy(weight_hbm.at[cmap_v], crows, sem).wait()
            for v in range(VOCAB):
                for j in range(D_MODEL // LANES):
                    tbl[v, pl.ds(j * LANES, LANES)] = wv[v, pl.ds(j * LANES, LANES)]
                    rev = lax.rev(
                        crows[v, pl.ds(D_MODEL - (j + 1) * LANES, LANES)],
                        dimensions=(0,),
                    )
                    tbl[v, pl.ds(D_MODEL + j * LANES, LANES)] = rev
            pltpu.sync_copy(tbl, tbl_hbm)

    return build(weight, cmap)


def _lookup(table, ids_flat, n_tokens):
    t_per_w = n_tokens // NW
    chunk = 32
    nbuf = 4
    n_chunks = t_per_w // chunk
    mesh = plsc.VectorSubcoreMesh(core_axis_name="c", subcore_axis_name="s")

    @functools.partial(
        pl.kernel,
        mesh=mesh,
        out_type=jax.ShapeDtypeStruct((n_tokens, D_OUT), jnp.float32),
        scratch_types=[
            pltpu.VMEM((t_per_w,), jnp.int32),
        ]
        + [pltpu.VMEM((chunk, D_OUT), jnp.float32) for _ in range(nbuf)]
        + [pltpu.SemaphoreType.DMA for _ in range(2 * nbuf)],
    )
    def look(tbl_hbm, ids_hbm, out_hbm, idx_v, *bufs_and_sems):
        rows = bufs_and_sems[:nbuf]
        gsem = bufs_and_sems[nbuf : 2 * nbuf]
        osem = bufs_and_sems[2 * nbuf :]
        wid = lax.axis_index("s") * NC + lax.axis_index("c")
        base = wid * t_per_w
        pltpu.sync_copy(ids_hbm.at[pl.ds(base, t_per_w)], idx_v)

        def gd(c, slot):
            return pltpu.make_async_copy(
                tbl_hbm.at[idx_v.at[pl.ds(c * chunk, chunk)]], rows[slot], gsem[slot]
            )

        def od(c, slot):
            return pltpu.make_async_copy(
                rows[slot], out_hbm.at[pl.ds(base + c * chunk, chunk)], osem[slot]
            )

        gd(0, 0).start()

        @pl.loop(0, n_chunks // nbuf)
        def _(g):
            for b in range(nbuf):
                c = g * nbuf + b
                nxt = c + 1
                slot_n = (b + 1) % nbuf

                @pl.when(nxt < n_chunks)
                def _():
                    @pl.when(nxt >= nbuf)
                    def _():
                        od(nxt - nbuf, slot_n).wait()

                    gd(nxt, slot_n).start()

                gd(c, b).wait()
                od(c, b).start()

        for b in range(nbuf):
            od(n_chunks - nbuf + b, b).wait()

    return look(table, ids_flat)


def kernel(input_ids, complement_map, weight):
    b, l = input_ids.shape
    n_tokens = b * l
    ids_flat = input_ids.reshape(n_tokens)
    table = _build_table(weight, complement_map)
    out = _lookup(table, ids_flat, n_tokens)
    return out.reshape(b, l, D_OUT)


# speedup vs baseline: 13.2518x; 3.0078x over previous
import functools

import jax
import jax.numpy as jnp
from jax import lax
from jax.experimental import pallas as pl
from jax.experimental.pallas import tpu as pltpu
from jax.experimental.pallas import tpu_sc as plsc

NC = 2
NS = 16
LANES = 16
NW = NC * NS

VOCAB = 16
D_MODEL = 256
D_OUT = 2 * D_MODEL


def _build_table(weight, cmap):
    mesh = plsc.VectorSubcoreMesh(core_axis_name="c", subcore_axis_name="s")

    @functools.partial(
        pl.kernel,
        mesh=mesh,
        out_type=jax.ShapeDtypeStruct((NW * VOCAB, D_OUT), jnp.float32),
        scratch_types=[
            pltpu.VMEM((VOCAB,), jnp.int32),
            pltpu.VMEM((VOCAB, D_MODEL), jnp.float32),
            pltpu.VMEM((VOCAB, D_MODEL), jnp.float32),
            pltpu.VMEM((VOCAB, D_OUT), jnp.float32),
            pltpu.SemaphoreType.DMA,
        ],
    )
    def build(weight_hbm, cmap_hbm, tbl_hbm, cmap_v, wv, crows, tbl, sem):
        wid = lax.axis_index("s") * NC + lax.axis_index("c")
        pltpu.sync_copy(weight_hbm, wv)
        pltpu.sync_copy(cmap_hbm, cmap_v)
        pltpu.async_copy(weight_hbm.at[cmap_v], crows, sem).wait()
        for v in range(VOCAB):
            for j in range(D_MODEL // LANES):
                tbl[v, pl.ds(j * LANES, LANES)] = wv[v, pl.ds(j * LANES, LANES)]
                rev = lax.rev(
                    crows[v, pl.ds(D_MODEL - (j + 1) * LANES, LANES)],
                    dimensions=(0,),
                )
                tbl[v, pl.ds(D_MODEL + j * LANES, LANES)] = rev
        pltpu.sync_copy(tbl, tbl_hbm.at[pl.ds(wid * VOCAB, VOCAB)])

    return build(weight, cmap)


def _lookup(table, ids_flat, n_tokens):
    t_per_w = n_tokens // NW
    chunk = 32
    nbuf = 4
    n_chunks = t_per_w // chunk
    mesh = plsc.VectorSubcoreMesh(core_axis_name="c", subcore_axis_name="s")

    @functools.partial(
        pl.kernel,
        mesh=mesh,
        out_type=jax.ShapeDtypeStruct((n_tokens, D_OUT), jnp.float32),
        scratch_types=[
            pltpu.VMEM((t_per_w,), jnp.int32),
        ]
        + [pltpu.VMEM((chunk, D_OUT), jnp.float32) for _ in range(nbuf)]
        + [pltpu.SemaphoreType.DMA for _ in range(2 * nbuf)],
    )
    def look(tbl_hbm, ids_hbm, out_hbm, idx_v, *bufs_and_sems):
        rows = bufs_and_sems[:nbuf]
        gsem = bufs_and_sems[nbuf : 2 * nbuf]
        osem = bufs_and_sems[2 * nbuf :]
        wid = lax.axis_index("s") * NC + lax.axis_index("c")
        base = wid * t_per_w
        pltpu.sync_copy(ids_hbm.at[pl.ds(base, t_per_w)], idx_v)

        off = wid * VOCAB

        @pl.loop(0, t_per_w // LANES)
        def _(i):
            idx_v[pl.ds(i * LANES, LANES)] = idx_v[pl.ds(i * LANES, LANES)] + off

        def gd(c, slot):
            return pltpu.make_async_copy(
                tbl_hbm.at[idx_v.at[pl.ds(c * chunk, chunk)]], rows[slot], gsem[slot]
            )

        def od(c, slot):
            return pltpu.make_async_copy(
                rows[slot], out_hbm.at[pl.ds(base + c * chunk, chunk)], osem[slot]
            )

        gd(0, 0).start()

        @pl.loop(0, n_chunks // nbuf)
        def _(g):
            for b in range(nbuf):
                c = g * nbuf + b
                nxt = c + 1
                slot_n = (b + 1) % nbuf

                @pl.when(nxt < n_chunks)
                def _():
                    @pl.when(nxt >= nbuf)
                    def _():
                        od(nxt - nbuf, slot_n).wait()

                    gd(nxt, slot_n).start()

                gd(c, b).wait()
                od(c, b).start()

        for b in range(nbuf):
            od(n_chunks - nbuf + b, b).wait()

    return look(table, ids_flat)


def kernel(input_ids, complement_map, weight):
    b, l = input_ids.shape
    n_tokens = b * l
    ids_flat = input_ids.reshape(n_tokens)
    table = _build_table(weight, complement_map)
    out = _lookup(table, ids_flat, n_tokens)
    return out.reshape(b, l, D_OUT)
